# Initial kernel scaffold; baseline (speedup 1.0000x reference)
import jax
import jax.numpy as jnp
from jax.experimental import pallas as pl


def kernel(edge_index, x, emb, W1, b1, W2, b2, P):
    raise NotImplementedError("write your pallas kernel here")


# speedup vs baseline: 3.3842x; 3.3842x over previous
import functools

import jax
import jax.numpy as jnp
from jax import lax
from jax.experimental import pallas as pl
from jax.experimental.pallas import tpu as pltpu
from jax.experimental.pallas import tpu_sc as plsc

N = 10000
E = 160000
F = 256
H = 512
C = 40
NP = 10240
FC = 128
NC, NS = 2, 16
EB = 128
NB = 80
NPT = NP // NS
RPT_S = NP // NS

_MESH = plsc.VectorSubcoreMesh(
    core_axis_name="c", subcore_axis_name="s", num_cores=NC, num_subcores=NS)


@functools.partial(
    pl.kernel,
    out_type=jax.ShapeDtypeStruct((NC * NS * NP,), jnp.float32),
    mesh=_MESH,
    compiler_params=pltpu.CompilerParams(needs_layout_passes=False),
    scratch_types=[
        pltpu.VMEM((NB, EB), jnp.int32),
        pltpu.VMEM((NP,), jnp.float32),
    ],
)
def _deg_hist(idx_hbm, out_hbm, idx_v, hist_v):
    c = lax.axis_index("c")
    s = lax.axis_index("s")
    w = c * NS + s

    pltpu.sync_copy(idx_hbm.at[w], idx_v)

    def zero(i, _):
        hist_v[pl.ds(i * 16, 16)] = jnp.zeros((16,), jnp.float32)
        return 0
    lax.fori_loop(0, NP // 16, zero, 0)

    ones = jnp.ones((16,), jnp.float32)

    def body(b, _):
        def inner(j, _):
            v = idx_v[b, pl.ds(j * 16, 16)]
            plsc.addupdate_scatter(hist_v, [v], ones)
            return 0
        lax.fori_loop(0, EB // 16, inner, 0)
        return 0
    lax.fori_loop(0, NB, body, 0)

    pltpu.sync_copy(hist_v, out_hbm.at[pl.ds(w * NP, NP)])


@functools.partial(
    pl.kernel,
    out_type=jax.ShapeDtypeStruct((NC * NP,), jnp.float32),
    mesh=_MESH,
    scratch_types=[
        pltpu.VMEM((NS, NPT), jnp.float32),
        pltpu.VMEM((NPT,), jnp.float32),
    ],
)
def _deg_reduce(parts_hbm, out_hbm, part_v, sum_v):
    c = lax.axis_index("c")
    s = lax.axis_index("s")
    off = s * NPT

    for j in range(NS):
        pltpu.sync_copy(parts_hbm.at[pl.ds((c * NS + j) * NP + off, NPT)],
                        part_v.at[j])

    def red(k, _):
        sl = pl.ds(k * 16, 16)
        acc = part_v[0, sl]
        for j in range(1, NS):
            acc = acc + part_v[j, sl]
        sum_v[sl] = acc
        return 0
    lax.fori_loop(0, NPT // 16, red, 0)

    pltpu.sync_copy(sum_v, out_hbm.at[pl.ds(c * NP + off, NPT)])


def _make_segsum(nchunk):
    passes = nchunk // NC

    @functools.partial(
        pl.kernel,
        out_type=jax.ShapeDtypeStruct((nchunk * NP, FC), jnp.float32),
        mesh=_MESH,
        scratch_types=[
            pltpu.VMEM((NB, EB), jnp.int32),
            pltpu.VMEM((NB, EB), jnp.int32),
            pltpu.VMEM((EB, FC), jnp.float32),
            pltpu.VMEM_SHARED((NP, FC), jnp.float32),
            pltpu.SemaphoreType.DMA,
        ],
    )
    def seg(sidx_hbm, didx_hbm, feat_hbm, out_hbm, sidx_v, didx_v, rows_v,
            acc, sem):
        c = lax.axis_index("c")
        s = lax.axis_index("s")

        pltpu.sync_copy(didx_hbm.at[s], didx_v)

        for p in range(passes):
            chunk = c * passes + p
            row_off = chunk * NP
            pltpu.sync_copy(sidx_hbm.at[s], sidx_v)

            def adj(r, _):
                for j in range(EB // 16):
                    sl = pl.ds(j * 16, 16)
                    sidx_v[r, sl] = sidx_v[r, sl] + row_off
                return 0
            lax.fori_loop(0, NB, adj, 0)

            def zrow(r, _):
                for j in range(FC // 16):
                    rows_v[r, pl.ds(j * 16, 16)] = jnp.zeros((16,),
                                                             jnp.float32)
                return 0
            lax.fori_loop(0, EB, zrow, 0)

            def zcp(j, _):
                pltpu.sync_copy(rows_v,
                                acc.at[pl.ds(s * RPT_S + j * 128, 128)])
                return 0
            lax.fori_loop(0, RPT_S // 128, zcp, 0)
            plsc.subcore_barrier()

            def body(b, _):
                pltpu.async_copy(feat_hbm.at[sidx_v.at[b]], rows_v,
                                 sem).wait()
                pltpu.sync_copy(rows_v, acc.at[didx_v.at[b]], add=True)
                return 0
            lax.fori_loop(0, NB, body, 0)
            plsc.subcore_barrier()

            def ocp(j, _):
                off = s * RPT_S + j * 128
                pltpu.sync_copy(acc.at[pl.ds(off, 128)], rows_v)
                pltpu.sync_copy(rows_v, out_hbm.at[pl.ds(chunk * NP + off,
                                                         128)])
                return 0
            lax.fori_loop(0, RPT_S // 128, ocp, 0)

    return seg


_segsum2 = _make_segsum(F // FC)
_segsum4 = _make_segsum(H // FC)


RB = 1280
GRID = NP // RB


def _prep_body(deg_ref, x_ref, o_ref):
    od = jnp.maximum(deg_ref[:, 0:1], 1.0)
    onorm = 1.0 / jnp.sqrt(od)
    xn = x_ref[...] * onorm
    o_ref[0] = xn[:, :FC]
    o_ref[1] = xn[:, FC:]


def _prep(deg, x):
    return pl.pallas_call(
        _prep_body,
        grid=(GRID,),
        in_specs=[
            pl.BlockSpec((RB, 2), lambda r: (r, 0)),
            pl.BlockSpec((RB, F), lambda r: (r, 0)),
        ],
        out_specs=pl.BlockSpec((2, RB, FC), lambda r: (0, r, 0)),
        out_shape=jax.ShapeDtypeStruct((2, NP, FC), jnp.float32),
    )(deg, x)


def _layer1_body(a0_ref, a1_ref, deg_ref, w_ref, b_ref, o_ref):
    od = jnp.maximum(deg_ref[:, 0:1], 1.0)
    idg = jnp.maximum(deg_ref[:, 1:2], 1.0)
    onorm = 1.0 / jnp.sqrt(od)
    inorm = 1.0 / jnp.sqrt(idg)
    acc = jnp.dot(a0_ref[...], w_ref[:FC, :],
                  preferred_element_type=jnp.float32)
    acc += jnp.dot(a1_ref[...], w_ref[FC:, :],
                   preferred_element_type=jnp.float32)
    h = jnp.maximum(acc * inorm + b_ref[...], 0.0) * onorm
    for k in range(H // FC):
        o_ref[k] = h[:, k * FC:(k + 1) * FC]


def _layer1(aggx, deg, W1, b1):
    return pl.pallas_call(
        _layer1_body,
        grid=(GRID,),
        in_specs=[
            pl.BlockSpec((RB, FC), lambda r: (r, 0)),
            pl.BlockSpec((RB, FC), lambda r: (r + GRID, 0)),
            pl.BlockSpec((RB, 2), lambda r: (r, 0)),
            pl.BlockSpec((F, H), lambda r: (0, 0)),
            pl.BlockSpec((1, H), lambda r: (0, 0)),
        ],
        out_specs=pl.BlockSpec((H // FC, RB, FC), lambda r: (0, r, 0)),
        out_shape=jax.ShapeDtypeStruct((H // FC, NP, FC), jnp.float32),
    )(aggx, aggx, deg, W1, b1)


def _head_body(a0_ref, a1_ref, a2_ref, a3_ref, deg_ref,
               w_ref, b_ref, emb_ref, pr_ref, o_ref):
    idg = jnp.maximum(deg_ref[:, 1:2], 1.0)
    inorm = 1.0 / jnp.sqrt(idg)
    a = (a0_ref, a1_ref, a2_ref, a3_ref)
    acc = jnp.dot(a[0][...], w_ref[:FC, :],
                  preferred_element_type=jnp.float32)
    for k in range(1, H // FC):
        acc += jnp.dot(a[k][...], w_ref[k * FC:(k + 1) * FC, :],
                       preferred_element_type=jnp.float32)
    h2 = acc * inorm + b_ref[...]
    e = emb_ref[...]
    pr = pr_ref[...]
    ssum = (jnp.sum(h2 * h2, axis=1, keepdims=True)
            + jnp.sum(e * e, axis=1, keepdims=True))
    zp = lax.dot_general(h2, pr[:, :H], (((1,), (1,)), ((), ())),
                         preferred_element_type=jnp.float32)
    zp += lax.dot_general(e, pr[:, H:], (((1,), (1,)), ((), ())),
                          preferred_element_type=jnp.float32)
    p2 = jnp.sum(pr * pr, axis=1)[None, :]
    o_ref[...] = -(ssum - 2.0 * zp + p2)


def _head(aggh, deg, W2, b2, emb, P):
    return pl.pallas_call(
        _head_body,
        grid=(GRID,),
        in_specs=[
            pl.BlockSpec((RB, FC), lambda r: (r, 0)),
            pl.BlockSpec((RB, FC), lambda r: (r + GRID, 0)),
            pl.BlockSpec((RB, FC), lambda r: (r + 2 * GRID, 0)),
            pl.BlockSpec((RB, FC), lambda r: (r + 3 * GRID, 0)),
            pl.BlockSpec((RB, 2), lambda r: (r, 0)),
            pl.BlockSpec((H, H), lambda r: (0, 0)),
            pl.BlockSpec((1, H), lambda r: (0, 0)),
            pl.BlockSpec((RB, H), lambda r: (r, 0)),
            pl.BlockSpec((C, 2 * H), lambda r: (0, 0)),
        ],
        out_specs=pl.BlockSpec((RB, C), lambda r: (r, 0)),
        out_shape=jax.ShapeDtypeStruct((NP, C), jnp.float32),
    )(aggh, aggh, aggh, aggh, deg, W2, b2, emb, P)


def kernel(edge_index, x, emb, W1, b1, W2, b2, P):
    src = edge_index[0]
    dst = edge_index[1]

    dstream = jnp.concatenate([src, dst]).reshape(NC * NS, -1)
    pad32 = jnp.full((NC * NS, NB * EB - dstream.shape[1]), N, jnp.int32)
    didx = jnp.concatenate([dstream, pad32], axis=1).reshape(NC * NS, NB, EB)

    padT = jnp.full((NS, NB * EB - E // NS), N, jnp.int32)
    sseg = jnp.concatenate([src.reshape(NS, -1), padT],
                           axis=1).reshape(NS, NB, EB)
    dseg = jnp.concatenate([dst.reshape(NS, -1), padT],
                           axis=1).reshape(NS, NB, EB)

    xp = jnp.zeros((NP, F), jnp.float32).at[:N].set(x)
    embp = jnp.zeros((NP, H), jnp.float32).at[:N].set(emb)

    parts = _deg_hist(didx)
    deg = _deg_reduce(parts).reshape(NC, NP).T
    xn = _prep(deg, xp).reshape(2 * NP, FC)
    aggx = _segsum2(sseg, dseg, xn)
    h1n = _layer1(aggx, deg, W1,
                  b1.reshape(1, H)).reshape(4 * NP, FC)
    aggh = _segsum4(sseg, dseg, h1n)
    return _head(aggh, deg, W2, b2.reshape(1, H), embp, P)[:N]
